# baseline (speedup 1.0000x reference)
import functools

import jax
import jax.numpy as jnp
from jax import lax
from jax.experimental import pallas as pl
from jax.experimental.pallas import tpu as pltpu
from jax.experimental.pallas import tpu_sc as plsc

D = 64
D2 = 128
NUM_REL = 100
K_PAD = 128
BATCH = 16384
TOT = 3 * BATCH
NC, NS = 2, 16
NW = NC * NS
ROWS_PER_W = TOT // NW
CHUNK = 128
NCHUNK = ROWS_PER_W // CHUNK
BB = 512
NBLK = BATCH // BB
M = NUM_REL * D
LAMBDA = 1e-05


def _sc_gather(table2, idx3):
    mesh = plsc.VectorSubcoreMesh(core_axis_name="c", subcore_axis_name="s")

    @functools.partial(
        pl.kernel, mesh=mesh,
        out_type=jax.ShapeDtypeStruct((TOT, D2), jnp.float32),
        scratch_types=[
            pltpu.VMEM((NCHUNK, CHUNK), jnp.int32),
            pltpu.VMEM((2, CHUNK, D2), jnp.float32),
            pltpu.SemaphoreType.DMA,
            pltpu.SemaphoreType.DMA,
        ],
    )
    def gath(table_hbm, idx_hbm, out_hbm, idx_v, rows_v, sem0, sem1):
        wid = lax.axis_index("s") * NC + lax.axis_index("c")
        base = wid * ROWS_PER_W
        pltpu.sync_copy(idx_hbm.at[wid], idx_v)
        sems = (sem0, sem1)
        copies = [None, None]
        copies[0] = pltpu.async_copy(
            table_hbm.at[idx_v.at[0]], rows_v.at[0], sems[0])
        for c in range(NCHUNK):
            nxt = (c + 1) % 2
            if c + 1 < NCHUNK:
                copies[nxt] = pltpu.async_copy(
                    table_hbm.at[idx_v.at[c + 1]], rows_v.at[nxt], sems[nxt])
            copies[c % 2].wait()
            pltpu.sync_copy(rows_v.at[c % 2],
                            out_hbm.at[pl.ds(base + c * CHUNK, CHUNK)])

    return gath(table2, idx3)


def _tc_loss(xcat, par2, r2, rel_pad, w_dup):
    def body(xh_ref, xp_ref, xn_ref, ph_ref, pp_ref, pn_ref,
             r_ref, rel_ref, w_ref, out_ref):
        i = pl.program_id(0)
        r = r_ref[...]
        mrow = lax.broadcasted_iota(jnp.int32, (BB, M), 1) // D
        mask = mrow == r
        srow = lax.broadcasted_iota(jnp.int32, (M, D), 0) % D
        scol = lax.broadcasted_iota(jnp.int32, (M, D), 1)
        sel = (srow == scol).astype(jnp.bfloat16)
        hi_lane = lax.broadcasted_iota(jnp.int32, (BB, D2), 1) >= D
        w = w_ref[...]

        def prod(x_ref, p_ref):
            par = p_ref[...] > 0
            keep = jnp.where(hi_lane == par, 1.0, 0.0)
            xb = (x_ref[...] * keep).astype(jnp.bfloat16)
            z = lax.dot_general(xb, w, (((1,), (0,)), ((), ())),
                                preferred_element_type=jnp.float32)
            zm = jnp.where(mask, z, 0.0).astype(jnp.bfloat16)
            return lax.dot_general(zm, sel, (((1,), (0,)), ((), ())),
                                   preferred_element_type=jnp.float32)

        rh = prod(xh_ref, ph_ref)
        rp = prod(xp_ref, pp_ref)
        rn = prod(xn_ref, pn_ref)
        kcol = lax.broadcasted_iota(jnp.int32, (BB, K_PAD), 1)
        onehot = (kcol == r).astype(jnp.float32)
        re = lax.dot_general(onehot, rel_ref[...], (((1,), (0,)), ((), ())),
                             preferred_element_type=jnp.float32)
        upos = rh + re - rp
        uneg = rh + re - rn
        pos = jnp.sum(upos * upos, axis=1, keepdims=True)
        neg = jnp.sum(uneg * uneg, axis=1, keepdims=True)
        x = pos - neg
        sp = jnp.maximum(x, 0.0) + jnp.log(1.0 + jnp.exp(-jnp.abs(x)))
        l2 = 0.5 * (jnp.sum(rh * rh, axis=1, keepdims=True)
                    + jnp.sum(re * re, axis=1, keepdims=True)
                    + jnp.sum(rp * rp, axis=1, keepdims=True)
                    + jnp.sum(rn * rn, axis=1, keepdims=True))
        tot = jnp.sum(sp + LAMBDA * l2)

        @pl.when(i == 0)
        def _init():
            out_ref[0, 0] = 0.0

        out_ref[0, 0] += tot

    fn = pl.pallas_call(
        body,
        grid=(NBLK,),
        in_specs=[
            pl.BlockSpec((BB, D2), lambda i: (i, 0)),
            pl.BlockSpec((BB, D2), lambda i: (i + NBLK, 0)),
            pl.BlockSpec((BB, D2), lambda i: (i + 2 * NBLK, 0)),
            pl.BlockSpec((BB, 1), lambda i: (i, 0)),
            pl.BlockSpec((BB, 1), lambda i: (i + NBLK, 0)),
            pl.BlockSpec((BB, 1), lambda i: (i + 2 * NBLK, 0)),
            pl.BlockSpec((BB, 1), lambda i: (i, 0)),
            pl.BlockSpec((K_PAD, D), lambda i: (0, 0)),
            pl.BlockSpec((D2, M), lambda i: (0, 0)),
        ],
        out_specs=pl.BlockSpec((1, 1), lambda i: (0, 0),
                               memory_space=pltpu.SMEM),
        out_shape=jax.ShapeDtypeStruct((1, 1), jnp.float32),
    )
    return fn(xcat, xcat, xcat, par2, par2, par2, r2, rel_pad, w_dup)


def kernel(h, r, pos_t, neg_t, entity_embed, relation_embed, W_R):
    idx = jnp.concatenate([h, pos_t, neg_t])
    table2 = entity_embed.reshape(500000, D2)
    idx3 = (idx >> 1).reshape(NW, NCHUNK, CHUNK)
    par2 = (idx & 1).reshape(TOT, 1)
    xcat = _sc_gather(table2, idx3)
    w_all = jnp.transpose(W_R, (1, 0, 2)).reshape(D, M)
    w_dup = jnp.concatenate([w_all, w_all], axis=0).astype(jnp.bfloat16)
    rel_pad = jnp.zeros((K_PAD, D), jnp.float32).at[:NUM_REL].set(relation_embed)
    acc = _tc_loss(xcat, par2, r.reshape(BATCH, 1), rel_pad, w_dup)
    return acc[0, 0] / BATCH


# speedup vs baseline: 1.0200x; 1.0200x over previous
import functools

import jax
import jax.numpy as jnp
from jax import lax
from jax.experimental import pallas as pl
from jax.experimental.pallas import tpu as pltpu
from jax.experimental.pallas import tpu_sc as plsc

D = 64
SUB = 8
N_ENT = 1000000
NUM_REL = 100
K_PAD = 128
BATCH = 16384
TOT = 3 * BATCH
NC, NS = 2, 16
NW = NC * NS
ROWS_PER_W = TOT // NW
CHUNK = 64
NCHUNK = ROWS_PER_W // CHUNK
BB = 512
NBLK = BATCH // BB
M = NUM_REL * D
LAMBDA = 1e-05


def _sc_gather(table, pidx3, lidx3):
    mesh = plsc.VectorSubcoreMesh(core_axis_name="c", subcore_axis_name="s")

    @functools.partial(
        pl.kernel, mesh=mesh,
        out_type=jax.ShapeDtypeStruct((TOT, D), jnp.float32),
        compiler_params=pltpu.CompilerParams(use_tc_tiling_on_sc=False),
        scratch_types=[
            pltpu.VMEM((NCHUNK, CHUNK), jnp.int32),
            pltpu.VMEM((2, CHUNK, D), jnp.float32),
            pltpu.SemaphoreType.DMA,
            pltpu.SemaphoreType.DMA,
        ],
    )
    def gath(table_hbm, pidx_hbm, lidx_hbm, out_hbm,
             pidx_v, rows_v, sem0, sem1):
        wid = lax.axis_index("s") * NC + lax.axis_index("c")
        base = wid * ROWS_PER_W
        pltpu.sync_copy(pidx_hbm.at[wid], pidx_v)
        sems = (sem0, sem1)
        g1 = [None, None]
        g1[0] = pltpu.async_copy(
            table_hbm.at[pidx_v.at[0]], rows_v.at[0], sems[0])
        for c in range(NCHUNK):
            buf = c % 2
            nxt = (c + 1) % 2
            if c + 1 < NCHUNK:
                g1[nxt] = pltpu.async_copy(
                    table_hbm.at[pidx_v.at[c + 1]], rows_v.at[nxt], sems[nxt])
            g1[buf].wait()
            pltpu.sync_copy(rows_v.at[buf],
                            out_hbm.at[pl.ds(base + c * CHUNK, CHUNK)])

    return gath(table, pidx3, lidx3)


def _tc_loss(xcat, r2, rel_pad, w_all):
    def body(xh_ref, xp_ref, xn_ref, r_ref, rel_ref, w_ref, out_ref):
        i = pl.program_id(0)
        r = r_ref[...]
        mrow = lax.broadcasted_iota(jnp.int32, (BB, M), 1) // D
        maskf = (mrow == r).astype(jnp.float32)
        srow = lax.broadcasted_iota(jnp.int32, (M, D), 0) % D
        scol = lax.broadcasted_iota(jnp.int32, (M, D), 1)
        sel = (srow == scol).astype(jnp.bfloat16)
        w = w_ref[...]

        def prod(x_ref):
            xb = x_ref[...].astype(jnp.bfloat16)
            z = lax.dot_general(xb, w, (((1,), (0,)), ((), ())),
                                preferred_element_type=jnp.float32)
            zm = (z * maskf).astype(jnp.bfloat16)
            return lax.dot_general(zm, sel, (((1,), (0,)), ((), ())),
                                   preferred_element_type=jnp.float32)

        rh = prod(xh_ref)
        rp = prod(xp_ref)
        rn = prod(xn_ref)
        kcol = lax.broadcasted_iota(jnp.int32, (BB, K_PAD), 1)
        onehot = (kcol == r).astype(jnp.float32)
        re = lax.dot_general(onehot, rel_ref[...], (((1,), (0,)), ((), ())),
                             preferred_element_type=jnp.float32)
        upos = rh + re - rp
        uneg = rh + re - rn
        pos = jnp.sum(upos * upos, axis=1, keepdims=True)
        neg = jnp.sum(uneg * uneg, axis=1, keepdims=True)
        x = pos - neg
        sp = jnp.maximum(x, 0.0) + jnp.log(1.0 + jnp.exp(-jnp.abs(x)))
        l2 = 0.5 * (jnp.sum(rh * rh, axis=1, keepdims=True)
                    + jnp.sum(re * re, axis=1, keepdims=True)
                    + jnp.sum(rp * rp, axis=1, keepdims=True)
                    + jnp.sum(rn * rn, axis=1, keepdims=True))
        tot = jnp.sum(sp + LAMBDA * l2)

        @pl.when(i == 0)
        def _init():
            out_ref[0, 0] = 0.0

        out_ref[0, 0] += tot

    fn = pl.pallas_call(
        body,
        grid=(NBLK,),
        in_specs=[
            pl.BlockSpec((BB, D), lambda i: (i, 0)),
            pl.BlockSpec((BB, D), lambda i: (i + NBLK, 0)),
            pl.BlockSpec((BB, D), lambda i: (i + 2 * NBLK, 0)),
            pl.BlockSpec((BB, 1), lambda i: (i, 0)),
            pl.BlockSpec((K_PAD, D), lambda i: (0, 0)),
            pl.BlockSpec((D, M), lambda i: (0, 0)),
        ],
        out_specs=pl.BlockSpec((1, 1), lambda i: (0, 0),
                               memory_space=pltpu.SMEM),
        out_shape=jax.ShapeDtypeStruct((1, 1), jnp.float32),
    )
    return fn(xcat, xcat, xcat, r2, rel_pad, w_all)


def kernel(h, r, pos_t, neg_t, entity_embed, relation_embed, W_R):
    idx = jnp.concatenate([h, pos_t, neg_t])
    pidx3 = idx.reshape(NW, NCHUNK, CHUNK)
    xcat = _sc_gather(entity_embed, pidx3, pidx3)
    w_all = jnp.transpose(W_R, (1, 0, 2)).reshape(D, M).astype(jnp.bfloat16)
    rel_pad = jnp.zeros((K_PAD, D), jnp.float32).at[:NUM_REL].set(relation_embed)
    acc = _tc_loss(xcat, r.reshape(BATCH, 1), rel_pad, w_all)
    return acc[0, 0] / BATCH


# speedup vs baseline: 1.1732x; 1.1503x over previous
import functools

import jax
import jax.numpy as jnp
from jax import lax
from jax.experimental import pallas as pl
from jax.experimental.pallas import tpu as pltpu
from jax.experimental.pallas import tpu_sc as plsc

D = 64
SUB = 8
N_ENT = 1000000
NUM_REL = 100
K_PAD = 128
BATCH = 16384
TOT = 3 * BATCH
NC, NS = 2, 16
NW = NC * NS
ROWS_PER_W = TOT // NW
CHUNK = 32
NCHUNK = ROWS_PER_W // CHUNK
BB = 512
NBLK = BATCH // BB
M = NUM_REL * D
LAMBDA = 1e-05


def _sc_gather(table, pidx3, lidx3):
    mesh = plsc.VectorSubcoreMesh(core_axis_name="c", subcore_axis_name="s")

    @functools.partial(
        pl.kernel, mesh=mesh,
        out_type=jax.ShapeDtypeStruct((TOT, D), jnp.float32),
        scratch_types=[
            pltpu.VMEM((NCHUNK, CHUNK), jnp.int32),
            pltpu.VMEM((16, SUB, D), jnp.float32),
            pltpu.VMEM((CHUNK, D), jnp.float32),
            pltpu.SemaphoreType.DMA,
        ],
    )
    def gath(table_hbm, pidx_hbm, lidx_hbm, out_hbm,
             idx_v, big_v, sel_v, sem0):
        wid = lax.axis_index("s") * NC + lax.axis_index("c")
        base = wid * ROWS_PER_W

        pltpu.sync_copy(pidx_hbm.at[wid], idx_v)

        def chunk_body(c, carry):
            def wave(w, carry2):
                vec = idx_v[c, pl.ds(w * 16, 16)]
                copies = []
                for j in range(16):
                    se = vec[j]
                    blk = pl.multiple_of((se >> 3) * SUB, SUB)
                    copies.append(pltpu.async_copy(
                        table_hbm.at[pl.ds(blk, SUB)], big_v.at[j], sem0))
                for cp in copies:
                    cp.wait()
                for j in range(16):
                    sub = vec[j] & (SUB - 1)
                    for g in range(D // 16):
                        sel_v[w * 16 + j, pl.ds(g * 16, 16)] = (
                            big_v[j, sub, pl.ds(g * 16, 16)])
                return carry2

            lax.fori_loop(0, CHUNK // 16, wave, 0)
            off = pl.multiple_of(base + c * CHUNK, 8)
            pltpu.sync_copy(sel_v, out_hbm.at[pl.ds(off, CHUNK)])
            return carry

        lax.fori_loop(0, NCHUNK, chunk_body, 0)

    return gath(table, pidx3, lidx3)


def _tc_loss(xcat, r2, rel_pad, w_all):
    def body(xh_ref, xp_ref, xn_ref, r_ref, rel_ref, w_ref, out_ref):
        i = pl.program_id(0)
        r = r_ref[...]
        mrow = lax.broadcasted_iota(jnp.int32, (BB, M), 1) // D
        maskf = (mrow == r).astype(jnp.float32)
        srow = lax.broadcasted_iota(jnp.int32, (M, D), 0) % D
        scol = lax.broadcasted_iota(jnp.int32, (M, D), 1)
        sel = (srow == scol).astype(jnp.bfloat16)
        w = w_ref[...]

        def prod(x_ref):
            xb = x_ref[...].astype(jnp.bfloat16)
            z = lax.dot_general(xb, w, (((1,), (0,)), ((), ())),
                                preferred_element_type=jnp.float32)
            zm = (z * maskf).astype(jnp.bfloat16)
            return lax.dot_general(zm, sel, (((1,), (0,)), ((), ())),
                                   preferred_element_type=jnp.float32)

        rh = prod(xh_ref)
        rp = prod(xp_ref)
        rn = prod(xn_ref)
        kcol = lax.broadcasted_iota(jnp.int32, (BB, K_PAD), 1)
        onehot = (kcol == r).astype(jnp.float32)
        re = lax.dot_general(onehot, rel_ref[...], (((1,), (0,)), ((), ())),
                             preferred_element_type=jnp.float32)
        upos = rh + re - rp
        uneg = rh + re - rn
        pos = jnp.sum(upos * upos, axis=1, keepdims=True)
        neg = jnp.sum(uneg * uneg, axis=1, keepdims=True)
        x = pos - neg
        sp = jnp.maximum(x, 0.0) + jnp.log(1.0 + jnp.exp(-jnp.abs(x)))
        l2 = 0.5 * (jnp.sum(rh * rh, axis=1, keepdims=True)
                    + jnp.sum(re * re, axis=1, keepdims=True)
                    + jnp.sum(rp * rp, axis=1, keepdims=True)
                    + jnp.sum(rn * rn, axis=1, keepdims=True))
        tot = jnp.sum(sp + LAMBDA * l2)

        @pl.when(i == 0)
        def _init():
            out_ref[0, 0] = 0.0

        out_ref[0, 0] += tot

    fn = pl.pallas_call(
        body,
        grid=(NBLK,),
        in_specs=[
            pl.BlockSpec((BB, D), lambda i: (i, 0)),
            pl.BlockSpec((BB, D), lambda i: (i + NBLK, 0)),
            pl.BlockSpec((BB, D), lambda i: (i + 2 * NBLK, 0)),
            pl.BlockSpec((BB, 1), lambda i: (i, 0)),
            pl.BlockSpec((K_PAD, D), lambda i: (0, 0)),
            pl.BlockSpec((D, M), lambda i: (0, 0)),
        ],
        out_specs=pl.BlockSpec((1, 1), lambda i: (0, 0),
                               memory_space=pltpu.SMEM),
        out_shape=jax.ShapeDtypeStruct((1, 1), jnp.float32),
    )
    return fn(xcat, xcat, xcat, r2, rel_pad, w_all)


def kernel(h, r, pos_t, neg_t, entity_embed, relation_embed, W_R):
    idx = jnp.concatenate([h, pos_t, neg_t])
    pidx3 = idx.reshape(NW, NCHUNK, CHUNK)
    xcat = _sc_gather(entity_embed, pidx3, pidx3)
    w_all = jnp.transpose(W_R, (1, 0, 2)).reshape(D, M).astype(jnp.bfloat16)
    rel_pad = jnp.zeros((K_PAD, D), jnp.float32).at[:NUM_REL].set(relation_embed)
    acc = _tc_loss(xcat, r.reshape(BATCH, 1), rel_pad, w_all)
    return acc[0, 0] / BATCH


# speedup vs baseline: 1.2550x; 1.0697x over previous
import functools

import jax
import jax.numpy as jnp
from jax import lax
from jax.experimental import pallas as pl
from jax.experimental.pallas import tpu as pltpu
from jax.experimental.pallas import tpu_sc as plsc

D = 64
SUB = 8
N_ENT = 1000000
NUM_REL = 100
K_PAD = 128
BATCH = 16384
TOT = 3 * BATCH
NC, NS = 2, 16
NW = NC * NS
ROWS_PER_W = TOT // NW
CHUNK = 32
NCHUNK = ROWS_PER_W // CHUNK
BB = 512
NBLK = BATCH // BB
M = NUM_REL * D
LAMBDA = 1e-05


def _sc_gather(table, pidx3, lidx3):
    mesh = plsc.VectorSubcoreMesh(core_axis_name="c", subcore_axis_name="s")

    @functools.partial(
        pl.kernel, mesh=mesh,
        out_type=jax.ShapeDtypeStruct((TOT, D), jnp.float32),
        scratch_types=[
            pltpu.VMEM((NCHUNK, CHUNK), jnp.int32),
            pltpu.VMEM((2, 16, SUB, D), jnp.float32),
            pltpu.VMEM((CHUNK, D), jnp.float32),
            pltpu.SemaphoreType.DMA,
            pltpu.SemaphoreType.DMA,
        ],
    )
    def gath(table_hbm, pidx_hbm, lidx_hbm, out_hbm,
             idx_v, big_v, sel_v, semA, semB):
        wid = lax.axis_index("s") * NC + lax.axis_index("c")
        base = wid * ROWS_PER_W

        pltpu.sync_copy(pidx_hbm.at[wid], idx_v)
        sems = (semA, semB)

        def fire(c, half):
            vec = idx_v[c, pl.ds(half * 16, 16)]
            for j in range(16):
                se = vec[j]
                blk = pl.multiple_of((se >> 3) * SUB, SUB)
                pltpu.async_copy(table_hbm.at[pl.ds(blk, SUB)],
                                 big_v.at[half, j], sems[half])
            return vec

        def drain(half):
            pltpu.make_async_copy(
                table_hbm.at[pl.ds(0, 16 * SUB)],
                big_v.at[half].reshape(16 * SUB, D),
                sems[half]).wait()

        def pick(c, half):
            vec = idx_v[c, pl.ds(half * 16, 16)]
            for j in range(16):
                sub = vec[j] & (SUB - 1)
                for g in range(D // 16):
                    sel_v[half * 16 + j, pl.ds(g * 16, 16)] = (
                        big_v[half, j, sub, pl.ds(g * 16, 16)])

        def flush(c):
            off = pl.multiple_of(base + c * CHUNK, 8)
            pltpu.sync_copy(sel_v, out_hbm.at[pl.ds(off, CHUNK)])

        fire(0, 0)

        def chunk_body(c, carry):
            fire(c, 1)
            drain(0)
            pick(c, 0)
            fire(c + 1, 0)
            drain(1)
            pick(c, 1)
            flush(c)
            return carry

        lax.fori_loop(0, NCHUNK - 1, chunk_body, 0)
        c_last = NCHUNK - 1
        fire(c_last, 1)
        drain(0)
        pick(c_last, 0)
        drain(1)
        pick(c_last, 1)
        flush(c_last)

    return gath(table, pidx3, lidx3)


def _tc_loss(xcat, r2, rel_pad, w_all):
    def body(xh_ref, xp_ref, xn_ref, r_ref, rel_ref, w_ref, out_ref):
        i = pl.program_id(0)
        r = r_ref[...]
        mrow = lax.broadcasted_iota(jnp.int32, (BB, M), 1) // D
        maskf = (mrow == r).astype(jnp.float32)
        srow = lax.broadcasted_iota(jnp.int32, (M, D), 0) % D
        scol = lax.broadcasted_iota(jnp.int32, (M, D), 1)
        sel = (srow == scol).astype(jnp.bfloat16)
        w = w_ref[...]

        def prod(x_ref):
            xb = x_ref[...].astype(jnp.bfloat16)
            z = lax.dot_general(xb, w, (((1,), (0,)), ((), ())),
                                preferred_element_type=jnp.float32)
            zm = (z * maskf).astype(jnp.bfloat16)
            return lax.dot_general(zm, sel, (((1,), (0,)), ((), ())),
                                   preferred_element_type=jnp.float32)

        rh = prod(xh_ref)
        rp = prod(xp_ref)
        rn = prod(xn_ref)
        kcol = lax.broadcasted_iota(jnp.int32, (BB, K_PAD), 1)
        onehot = (kcol == r).astype(jnp.float32)
        re = lax.dot_general(onehot, rel_ref[...], (((1,), (0,)), ((), ())),
                             preferred_element_type=jnp.float32)
        upos = rh + re - rp
        uneg = rh + re - rn
        pos = jnp.sum(upos * upos, axis=1, keepdims=True)
        neg = jnp.sum(uneg * uneg, axis=1, keepdims=True)
        x = pos - neg
        sp = jnp.maximum(x, 0.0) + jnp.log(1.0 + jnp.exp(-jnp.abs(x)))
        l2 = 0.5 * (jnp.sum(rh * rh, axis=1, keepdims=True)
                    + jnp.sum(re * re, axis=1, keepdims=True)
                    + jnp.sum(rp * rp, axis=1, keepdims=True)
                    + jnp.sum(rn * rn, axis=1, keepdims=True))
        tot = jnp.sum(sp + LAMBDA * l2)

        @pl.when(i == 0)
        def _init():
            out_ref[0, 0] = 0.0

        out_ref[0, 0] += tot

    fn = pl.pallas_call(
        body,
        grid=(NBLK,),
        in_specs=[
            pl.BlockSpec((BB, D), lambda i: (i, 0)),
            pl.BlockSpec((BB, D), lambda i: (i + NBLK, 0)),
            pl.BlockSpec((BB, D), lambda i: (i + 2 * NBLK, 0)),
            pl.BlockSpec((BB, 1), lambda i: (i, 0)),
            pl.BlockSpec((K_PAD, D), lambda i: (0, 0)),
            pl.BlockSpec((D, M), lambda i: (0, 0)),
        ],
        out_specs=pl.BlockSpec((1, 1), lambda i: (0, 0),
                               memory_space=pltpu.SMEM),
        out_shape=jax.ShapeDtypeStruct((1, 1), jnp.float32),
    )
    return fn(xcat, xcat, xcat, r2, rel_pad, w_all)


def kernel(h, r, pos_t, neg_t, entity_embed, relation_embed, W_R):
    idx = jnp.concatenate([h, pos_t, neg_t])
    pidx3 = idx.reshape(NW, NCHUNK, CHUNK)
    xcat = _sc_gather(entity_embed, pidx3, pidx3)
    w_all = jnp.transpose(W_R, (1, 0, 2)).reshape(D, M).astype(jnp.bfloat16)
    rel_pad = jnp.zeros((K_PAD, D), jnp.float32).at[:NUM_REL].set(relation_embed)
    acc = _tc_loss(xcat, r.reshape(BATCH, 1), rel_pad, w_all)
    return acc[0, 0] / BATCH
